# baseline (speedup 1.0000x reference)
import functools

import jax
import jax.numpy as jnp
from jax import lax
from jax.experimental import pallas as pl
from jax.experimental.pallas import tpu as pltpu
from jax.experimental.pallas import tpu_sc as plsc

N = 10000
E = 320000
X = 128
H = 128
C = 10
DELTA = 2.5

NC = 2
NS = 16
NW = NC * NS
TILE_N = 320
NPAD = NW * TILE_N
CAP = 16384
CE = 4000
CS = 112
HL = H // 16

_ROWS = 1000


def _route_body(esrc_hbm, edst_hbm, selpk_hbm, counts_hbm, cntf_hbm,
                src_v, dst_v, selpk_v, sorted_v, cvec_v, hist_fv,
                hist_s, start_s, esem):
    w = lax.axis_index("s") * NC + lax.axis_index("c")
    lo = w * TILE_N
    zero16 = jnp.zeros((16,), jnp.int32)
    sent16 = jnp.full((16,), TILE_N, jnp.int32)
    NG = CE // 16

    def fire_e(c, b):
        pltpu.async_copy(esrc_hbm.at[pl.ds(c * CE, CE)],
                         src_v.at[pl.ds(b * CE, CE)], esem.at[b])
        pltpu.async_copy(edst_hbm.at[pl.ds(c * CE, CE)],
                         dst_v.at[pl.ds(b * CE, CE)], esem.at[b])

    def wait_e(c, b):
        pltpu.make_async_copy(esrc_hbm.at[pl.ds(c * CE, CE)],
                              src_v.at[pl.ds(b * CE, CE)], esem.at[b]).wait()
        pltpu.make_async_copy(edst_hbm.at[pl.ds(c * CE, CE)],
                              dst_v.at[pl.ds(b * CE, CE)], esem.at[b]).wait()

    fire_e(0, 0)

    def chunk(cb, off_v):
        b = cb % 2

        @pl.when(cb + 1 < E // CE)
        def _():
            fire_e(cb + 1, 1 - b)

        wait_e(cb, b)

        def step(g, off_v):
            dl = dst_v[pl.ds(b * CE + g * 16, 16)] - lo
            msk = (dl >= 0) & (dl < TILE_N)
            npop = plsc.all_reduce_population_count(msk)

            def sel(off_v):
                s16 = src_v[pl.ds(b * CE + g * 16, 16)]
                pos = plsc.cumsum(jnp.where(msk, 1, 0))
                idx = off_v + pos - 1
                val = jnp.left_shift(s16, 9) | (dl & 511)
                plsc.store_scatter(selpk_v, [idx], val, mask=msk)
                return off_v + npop

            def skip(off_v):
                return off_v

            return lax.cond(npop[0] > 0, sel, skip, off_v)

        return lax.fori_loop(0, NG, step, off_v)

    off_v = lax.fori_loop(0, E // CE, chunk, zero16)
    count = off_v[0]

    def pad(i, _):
        selpk_v[pl.ds(count + i * 16, 16)] = sent16
        return 0

    lax.fori_loop(0, CS // 16, pad, 0)

    cvec_v[...] = zero16 + count
    pltpu.sync_copy(cvec_v, counts_hbm.at[w])

    def hzero(i, _):
        hist_s[i] = 0
        return 0

    lax.fori_loop(0, TILE_N + 1, hzero, 0)

    ngrp = ((count + CS - 1) // CS) * (CS // 16)

    def hstep(gi, _):
        v = selpk_v[pl.ds(gi * 16, 16)]
        for l in range(16):
            d = v[l] & 511
            hist_s[d] = hist_s[d] + 1
        return 0

    lax.fori_loop(0, ngrp, hstep, 0)

    lanes = lax.iota(jnp.int32, 16)
    lane0 = lanes == 0

    def hout(i, _):
        hv = (zero16 + hist_s[i]).astype(jnp.float32)
        plsc.store_scatter(hist_fv, [zero16 + i], hv, mask=lane0)
        return 0

    lax.fori_loop(0, TILE_N, hout, 0)
    pltpu.sync_copy(hist_fv, cntf_hbm.at[w])

    def pfx(d, run):
        start_s[d] = run
        return run + hist_s[d]

    lax.fori_loop(0, TILE_N + 1, pfx, jnp.int32(0))

    def rstep(gi, _):
        v = selpk_v[pl.ds(gi * 16, 16)]
        for l in range(16):
            pk = v[l]
            d = pk & 511
            slot = start_s[d]
            start_s[d] = slot + 1
            plsc.store_scatter(sorted_v, [zero16 + slot], zero16 + pk,
                               mask=lane0)
        return 0

    lax.fori_loop(0, (count + 15) // 16, rstep, 0)

    def pad2(i, _):
        sorted_v[pl.ds(count + i * 16, 16)] = sent16
        return 0

    lax.fori_loop(0, CS // 16, pad2, 0)
    pltpu.sync_copy(sorted_v, selpk_hbm.at[w])


def _route(esrc, edst):
    mesh = plsc.VectorSubcoreMesh(core_axis_name="c", subcore_axis_name="s")
    f = pl.kernel(
        _route_body,
        out_type=[
            jax.ShapeDtypeStruct((NW, CAP), jnp.int32),
            jax.ShapeDtypeStruct((NW, 16), jnp.int32),
            jax.ShapeDtypeStruct((NW, TILE_N), jnp.float32),
        ],
        mesh=mesh,
        scratch_types=[
            pltpu.VMEM((2 * CE,), jnp.int32),
            pltpu.VMEM((2 * CE,), jnp.int32),
            pltpu.VMEM((CAP,), jnp.int32),
            pltpu.VMEM((CAP,), jnp.int32),
            pltpu.VMEM((16,), jnp.int32),
            pltpu.VMEM((TILE_N,), jnp.float32),
            pltpu.SMEM((TILE_N + 8,), jnp.int32),
            pltpu.SMEM((TILE_N + 8,), jnp.int32),
            pltpu.SemaphoreType.DMA((2,)),
        ],
        compiler_params=pltpu.CompilerParams(needs_layout_passes=False),
    )
    return f(esrc, edst)


def _agg_body(m_hbm, selpk_hbm, counts_hbm, sum_hbm, max_hbm,
              pkall_v, src_v, rows_v, cvec_v, acc_sum, acc_max, gsem):
    w = lax.axis_index("s") * NC + lax.axis_index("c")
    z16 = jnp.zeros((16,), jnp.float32)

    def zr(i, _):
        acc_sum[pl.ds(i * 16, 16)] = z16
        acc_max[pl.ds(i * 16, 16)] = z16
        return 0

    lax.fori_loop(0, (TILE_N + 1) * H // 16, zr, 0)

    pltpu.sync_copy(selpk_hbm.at[w], pkall_v)
    pltpu.sync_copy(counts_hbm.at[w], cvec_v)
    count = cvec_v[...][0]
    nch = (count + CS - 1) // CS

    def fire(c, b):
        def up(g, _):
            v = pkall_v[pl.ds(c * CS + g * 16, 16)]
            src_v[b, pl.ds(g * 16, 16)] = jnp.right_shift(v, 9)
            return 0

        lax.fori_loop(0, CS // 16, up, 0)
        pltpu.async_copy(m_hbm.at[src_v.at[b]], rows_v.at[b], gsem.at[b])

    def wait_g(b):
        pltpu.make_async_copy(m_hbm.at[src_v.at[b]], rows_v.at[b],
                              gsem.at[b]).wait()

    @pl.when(nch > 0)
    def _():
        fire(0, 0)

    def flush(cur_d, regs):
        off = cur_d * H
        for j in range(HL):
            acc_sum[pl.ds(off + j * 16, 16)] = regs[j]
            acc_max[pl.ds(off + j * 16, 16)] = regs[HL + j]

    z16f = jnp.zeros((16,), jnp.float32)
    zregs = (z16f,) * (2 * HL)

    def chunk(cb, carry):
        b = cb % 2

        @pl.when(cb + 1 < nch)
        def _():
            fire(cb + 1, 1 - b)

        wait_g(b)

        def grp(gl, carry):
            dlv = pkall_v[pl.ds(cb * CS + gl * 16, 16)]
            for l in range(16):
                d = dlv[l] & 511
                cur_d, regs = carry[0], carry[1:]

                def on_change(cd, rs):
                    flush(cd, rs)
                    return (d,) + zregs

                def keep(cd, rs):
                    return (cd,) + rs

                carry = lax.cond(d != cur_d, on_change, keep, cur_d, regs)
                row = [rows_v[b, gl * 16 + l, pl.ds(j * 16, 16)]
                       for j in range(HL)]
                carry = (carry[0],) + tuple(
                    carry[1 + j] + row[j] for j in range(HL)) + tuple(
                    jnp.maximum(carry[1 + HL + j], row[j]) for j in range(HL))
            return carry

        return lax.fori_loop(0, CS // 16, grp, carry)

    fcarry = lax.fori_loop(0, nch, chunk, (jnp.int32(TILE_N),) + zregs)
    flush(fcarry[0], fcarry[1:])

    pltpu.sync_copy(acc_sum.at[pl.ds(0, TILE_N * H)],
                    sum_hbm.at[pl.ds(w * TILE_N * H, TILE_N * H)])
    pltpu.sync_copy(acc_max.at[pl.ds(0, TILE_N * H)],
                    max_hbm.at[pl.ds(w * TILE_N * H, TILE_N * H)])


def _agg(m, sel_pk, counts):
    mesh = plsc.VectorSubcoreMesh(core_axis_name="c", subcore_axis_name="s")
    f = pl.kernel(
        _agg_body,
        out_type=[
            jax.ShapeDtypeStruct((NPAD * H,), jnp.float32),
            jax.ShapeDtypeStruct((NPAD * H,), jnp.float32),
        ],
        mesh=mesh,
        scratch_types=[
            pltpu.VMEM((CAP,), jnp.int32),
            pltpu.VMEM((2, CS), jnp.int32),
            pltpu.VMEM((2, CS, H), jnp.float32),
            pltpu.VMEM((16,), jnp.int32),
            pltpu.VMEM(((TILE_N + 1) * H,), jnp.float32),
            pltpu.VMEM(((TILE_N + 1) * H,), jnp.float32),
            pltpu.SemaphoreType.DMA((2,)),
        ],
        compiler_params=pltpu.CompilerParams(needs_layout_passes=False),
    )
    return f(m, sel_pk, counts)


def _mlp1_body(x_ref, w1_ref, b1_ref, w2_ref, b2_ref, w3_ref, b3_ref,
               wm_ref, bm_ref, feats_ref, m0_ref):
    h = jax.nn.relu(x_ref[...] @ w1_ref[...] + b1_ref[...])
    h = jax.nn.relu(h @ w2_ref[...] + b2_ref[...])
    f = h @ w3_ref[...] + b3_ref[...]
    feats_ref[...] = f
    wm = wm_ref[...]
    m0_ref[...] = jax.nn.relu(f @ (wm[:H, :] + wm[H:, :]) + bm_ref[...])


def _mlp1(x, w1, b1, w2, b2, w3, b3, wm, bm):
    grid = (N // _ROWS,)
    row_spec = pl.BlockSpec((_ROWS, H), lambda i: (i, 0))
    full = lambda a: pl.BlockSpec(a.shape, lambda i: (0,) * a.ndim)
    return pl.pallas_call(
        _mlp1_body,
        grid=grid,
        in_specs=[pl.BlockSpec((_ROWS, X), lambda i: (i, 0)),
                  full(w1), full(b1), full(w2), full(b2), full(w3), full(b3),
                  full(wm), full(bm)],
        out_specs=[row_spec, row_spec],
        out_shape=[jax.ShapeDtypeStruct((N, H), jnp.float32),
                   jax.ShapeDtypeStruct((N, H), jnp.float32)],
    )(x, w1, b1, w2, b2, w3, b3, wm, bm)


def _upd_common(res_ref, s_ref, mx_ref, cnt_ref, wu_ref, bu_ref):
    res = res_ref[...]
    s = s_ref[...]
    mx = mx_ref[...]
    cnt = cnt_ref[...]
    mean = s / jnp.maximum(cnt, 1.0)
    amp = jnp.log(cnt + 1.0) / DELTA
    wu = wu_ref[...]
    acc = (res @ wu[0:H, :] + mean @ wu[H:2 * H, :] + mx @ wu[2 * H:3 * H, :]
           + s @ wu[3 * H:4 * H, :] + (mean * amp) @ wu[4 * H:5 * H, :]
           + (mx * amp) @ wu[5 * H:6 * H, :] + (s * amp) @ wu[6 * H:7 * H, :]
           + bu_ref[...])
    return acc + res


def _upd0_body(res_ref, s_ref, mx_ref, cnt_ref, wu_ref, bu_ref,
               feats_ref, wm_ref, bm_ref, out_ref, m1_ref):
    out = _upd_common(res_ref, s_ref, mx_ref, cnt_ref, wu_ref, bu_ref)
    out_ref[...] = out
    wm = wm_ref[...]
    m1_ref[...] = jax.nn.relu(
        out @ wm[:H, :] + feats_ref[...] @ wm[H:, :] + bm_ref[...])


def _upd1_body(res_ref, s_ref, mx_ref, cnt_ref, wu_ref, bu_ref, out_ref):
    out_ref[...] = _upd_common(res_ref, s_ref, mx_ref, cnt_ref, wu_ref, bu_ref)


def _update0(res, s, mx, cnt, wu, bu, feats, wm, bm):
    grid = (N // _ROWS,)
    row_spec = pl.BlockSpec((_ROWS, H), lambda i: (i, 0))
    full = lambda a: pl.BlockSpec(a.shape, lambda i: (0,) * a.ndim)
    return pl.pallas_call(
        _upd0_body,
        grid=grid,
        in_specs=[row_spec, row_spec, row_spec,
                  pl.BlockSpec((_ROWS, 1), lambda i: (i, 0)),
                  full(wu), full(bu), row_spec, full(wm), full(bm)],
        out_specs=[row_spec, row_spec],
        out_shape=[jax.ShapeDtypeStruct((N, H), jnp.float32),
                   jax.ShapeDtypeStruct((N, H), jnp.float32)],
    )(res, s, mx, cnt, wu, bu, feats, wm, bm)


def _update1(res, s, mx, cnt, wu, bu):
    grid = (N // _ROWS,)
    row_spec = pl.BlockSpec((_ROWS, H), lambda i: (i, 0))
    full = lambda a: pl.BlockSpec(a.shape, lambda i: (0,) * a.ndim)
    return pl.pallas_call(
        _upd1_body,
        grid=grid,
        in_specs=[row_spec, row_spec, row_spec,
                  pl.BlockSpec((_ROWS, 1), lambda i: (i, 0)),
                  full(wu), full(bu)],
        out_specs=row_spec,
        out_shape=jax.ShapeDtypeStruct((N, H), jnp.float32),
    )(res, s, mx, cnt, wu, bu)


def _pool_body(res_ref, w_ref, b_ref, out_ref):
    pooled = jnp.sum(res_ref[...], axis=0, keepdims=True)
    out_ref[...] = pooled @ w_ref[...] + b_ref[...]


def _pool(res, w, b):
    return pl.pallas_call(
        _pool_body,
        out_shape=jax.ShapeDtypeStruct((1, C), jnp.float32),
    )(res, w, b)


def kernel(x, edge_index,
           mlp1_W1, mlp1_b1, mlp1_W2, mlp1_b2, mlp1_W3, mlp1_b3,
           conv0_Wm, conv0_bm, conv0_Wu, conv0_bu,
           conv1_Wm, conv1_bm, conv1_Wu, conv1_bu,
           lin_W, lin_b):
    b1 = mlp1_b1.reshape(1, -1)
    b2 = mlp1_b2.reshape(1, -1)
    b3 = mlp1_b3.reshape(1, -1)
    bm0 = conv0_bm.reshape(1, -1)
    bu0 = conv0_bu.reshape(1, -1)
    bm1 = conv1_bm.reshape(1, -1)
    bu1 = conv1_bu.reshape(1, -1)
    lb = lin_b.reshape(1, -1)

    feats, m0 = _mlp1(x, mlp1_W1, b1, mlp1_W2, b2, mlp1_W3, b3,
                      conv0_Wm, bm0)

    sel_pk, counts, cnt_f = _route(edge_index[0], edge_index[1])
    cnt = cnt_f.reshape(-1)[:N].reshape(N, 1)

    s0, mx0 = _agg(m0, sel_pk, counts)
    s0 = s0.reshape(NPAD, H)[:N]
    mx0 = mx0.reshape(NPAD, H)[:N]
    res1, m1 = _update0(feats, s0, mx0, cnt, conv0_Wu, bu0,
                        feats, conv1_Wm, bm1)

    s1, mx1 = _agg(m1, sel_pk, counts)
    s1 = s1.reshape(NPAD, H)[:N]
    mx1 = mx1.reshape(NPAD, H)[:N]
    res2 = _update1(res1, s1, mx1, cnt, conv1_Wu, bu1)

    return _pool(res2, lin_W, lb)


# speedup vs baseline: 1.4592x; 1.4592x over previous
import functools

import jax
import jax.numpy as jnp
from jax import lax
from jax.experimental import pallas as pl
from jax.experimental.pallas import tpu as pltpu
from jax.experimental.pallas import tpu_sc as plsc

N = 10000
E = 320000
X = 128
H = 128
C = 10
DELTA = 2.5

NC = 2
NS = 16
NW = NC * NS
TILE_N = 320
NPAD = NW * TILE_N
CAP = 16384
CE = 4000
CS = 112
HL = H // 16

_ROWS = 1000


def _route_body(esrc_hbm, edst_hbm, selpk_hbm, counts_hbm, cntf_hbm,
                src_v, dst_v, selpk_v, sorted_v, cvec_v, hist_fv,
                hist_s, start_s, esem):
    w = lax.axis_index("s") * NC + lax.axis_index("c")
    lo = w * TILE_N
    zero16 = jnp.zeros((16,), jnp.int32)
    sent16 = jnp.full((16,), TILE_N, jnp.int32)
    NG = CE // 16

    def fire_e(c, b):
        pltpu.async_copy(esrc_hbm.at[pl.ds(c * CE, CE)],
                         src_v.at[pl.ds(b * CE, CE)], esem.at[b])
        pltpu.async_copy(edst_hbm.at[pl.ds(c * CE, CE)],
                         dst_v.at[pl.ds(b * CE, CE)], esem.at[b])

    def wait_e(c, b):
        pltpu.make_async_copy(esrc_hbm.at[pl.ds(c * CE, CE)],
                              src_v.at[pl.ds(b * CE, CE)], esem.at[b]).wait()
        pltpu.make_async_copy(edst_hbm.at[pl.ds(c * CE, CE)],
                              dst_v.at[pl.ds(b * CE, CE)], esem.at[b]).wait()

    fire_e(0, 0)

    def chunk(cb, off_v):
        b = cb % 2

        @pl.when(cb + 1 < E // CE)
        def _():
            fire_e(cb + 1, 1 - b)

        wait_e(cb, b)

        def step(g, off_v):
            base = b * CE + g * 32
            dla = dst_v[pl.ds(base, 16)] - lo
            dlb = dst_v[pl.ds(base + 16, 16)] - lo
            sa = src_v[pl.ds(base, 16)]
            sb = src_v[pl.ds(base + 16, 16)]
            ma = (dla >= 0) & (dla < TILE_N)
            mb = (dlb >= 0) & (dlb < TILE_N)
            pa = plsc.cumsum(jnp.where(ma, 1, 0))
            pb = plsc.cumsum(jnp.where(mb, 1, 0))
            na = plsc.all_reduce_population_count(ma)
            nb = plsc.all_reduce_population_count(mb)
            va = jnp.left_shift(sa, 9) | (dla & 511)
            vb = jnp.left_shift(sb, 9) | (dlb & 511)
            plsc.store_scatter(selpk_v, [off_v + pa - 1], va, mask=ma)
            plsc.store_scatter(selpk_v, [off_v + na + pb - 1], vb, mask=mb)
            return off_v + na + nb

        return lax.fori_loop(0, NG // 2, step, off_v)

    off_v = lax.fori_loop(0, E // CE, chunk, zero16)
    count = off_v[0]

    def pad(i, _):
        selpk_v[pl.ds(count + i * 16, 16)] = sent16
        return 0

    lax.fori_loop(0, CS // 16, pad, 0)

    cvec_v[...] = zero16 + count
    pltpu.sync_copy(cvec_v, counts_hbm.at[w])

    def hzero(i, _):
        hist_s[i] = 0
        return 0

    lax.fori_loop(0, TILE_N + 1, hzero, 0)

    ngrp = ((count + CS - 1) // CS) * (CS // 16)

    def hstep(gi, _):
        v = selpk_v[pl.ds(gi * 16, 16)]
        for l in range(16):
            d = v[l] & 511
            hist_s[d] = hist_s[d] + 1
        return 0

    lax.fori_loop(0, ngrp, hstep, 0)

    lanes = lax.iota(jnp.int32, 16)
    lane0 = lanes == 0

    def hout(i, _):
        hv = (zero16 + hist_s[i]).astype(jnp.float32)
        plsc.store_scatter(hist_fv, [zero16 + i], hv, mask=lane0)
        return 0

    lax.fori_loop(0, TILE_N, hout, 0)
    pltpu.sync_copy(hist_fv, cntf_hbm.at[w])

    def pfx(d, run):
        start_s[d] = run
        return run + hist_s[d]

    lax.fori_loop(0, TILE_N + 1, pfx, jnp.int32(0))

    def rstep(gi, _):
        v = selpk_v[pl.ds(gi * 16, 16)]
        for l in range(16):
            pk = v[l]
            d = pk & 511
            slot = start_s[d]
            start_s[d] = slot + 1
            plsc.store_scatter(sorted_v, [zero16 + slot], zero16 + pk,
                               mask=lane0)
        return 0

    lax.fori_loop(0, (count + 15) // 16, rstep, 0)

    def pad2(i, _):
        sorted_v[pl.ds(count + i * 16, 16)] = sent16
        return 0

    lax.fori_loop(0, CS // 16, pad2, 0)
    pltpu.sync_copy(sorted_v, selpk_hbm.at[w])


def _route(esrc, edst):
    mesh = plsc.VectorSubcoreMesh(core_axis_name="c", subcore_axis_name="s")
    f = pl.kernel(
        _route_body,
        out_type=[
            jax.ShapeDtypeStruct((NW, CAP), jnp.int32),
            jax.ShapeDtypeStruct((NW, 16), jnp.int32),
            jax.ShapeDtypeStruct((NW, TILE_N), jnp.float32),
        ],
        mesh=mesh,
        scratch_types=[
            pltpu.VMEM((2 * CE,), jnp.int32),
            pltpu.VMEM((2 * CE,), jnp.int32),
            pltpu.VMEM((CAP,), jnp.int32),
            pltpu.VMEM((CAP,), jnp.int32),
            pltpu.VMEM((16,), jnp.int32),
            pltpu.VMEM((TILE_N,), jnp.float32),
            pltpu.SMEM((TILE_N + 8,), jnp.int32),
            pltpu.SMEM((TILE_N + 8,), jnp.int32),
            pltpu.SemaphoreType.DMA((2,)),
        ],
        compiler_params=pltpu.CompilerParams(needs_layout_passes=False),
    )
    return f(esrc, edst)


def _agg_body(m_hbm, selpk_hbm, counts_hbm, sum_hbm, max_hbm,
              pkall_v, src_v, rows_v, cvec_v, acc_sum, acc_max, gsem):
    w = lax.axis_index("s") * NC + lax.axis_index("c")
    z16 = jnp.zeros((16,), jnp.float32)

    def zr(i, _):
        acc_sum[pl.ds(i * 16, 16)] = z16
        acc_max[pl.ds(i * 16, 16)] = z16
        return 0

    lax.fori_loop(0, (TILE_N + 1) * H // 16, zr, 0)

    pltpu.sync_copy(selpk_hbm.at[w], pkall_v)
    pltpu.sync_copy(counts_hbm.at[w], cvec_v)
    count = cvec_v[...][0]
    nch = (count + CS - 1) // CS

    def fire(c, b):
        def up(g, _):
            v = pkall_v[pl.ds(c * CS + g * 16, 16)]
            src_v[b, pl.ds(g * 16, 16)] = jnp.right_shift(v, 9)
            return 0

        lax.fori_loop(0, CS // 16, up, 0)
        pltpu.async_copy(m_hbm.at[src_v.at[b]], rows_v.at[b], gsem.at[b])

    def wait_g(b):
        pltpu.make_async_copy(m_hbm.at[src_v.at[b]], rows_v.at[b],
                              gsem.at[b]).wait()

    @pl.when(nch > 0)
    def _():
        fire(0, 0)

    def flush(cur_d, regs):
        off = cur_d * H
        for j in range(HL):
            acc_sum[pl.ds(off + j * 16, 16)] = regs[j]
            acc_max[pl.ds(off + j * 16, 16)] = regs[HL + j]

    z16f = jnp.zeros((16,), jnp.float32)
    zregs = (z16f,) * (2 * HL)

    def chunk(cb, carry):
        b = cb % 2

        @pl.when(cb + 1 < nch)
        def _():
            fire(cb + 1, 1 - b)

        wait_g(b)

        def grp(gl, carry):
            dlv = pkall_v[pl.ds(cb * CS + gl * 16, 16)]
            for l in range(16):
                d = dlv[l] & 511
                cur_d, regs = carry[0], carry[1:]

                def on_change(cd, rs):
                    flush(cd, rs)
                    return (d,) + zregs

                def keep(cd, rs):
                    return (cd,) + rs

                carry = lax.cond(d != cur_d, on_change, keep, cur_d, regs)
                row = [rows_v[b, gl * 16 + l, pl.ds(j * 16, 16)]
                       for j in range(HL)]
                carry = (carry[0],) + tuple(
                    carry[1 + j] + row[j] for j in range(HL)) + tuple(
                    jnp.maximum(carry[1 + HL + j], row[j]) for j in range(HL))
            return carry

        return lax.fori_loop(0, CS // 16, grp, carry)

    fcarry = lax.fori_loop(0, nch, chunk, (jnp.int32(TILE_N),) + zregs)
    flush(fcarry[0], fcarry[1:])

    pltpu.sync_copy(acc_sum.at[pl.ds(0, TILE_N * H)],
                    sum_hbm.at[pl.ds(w * TILE_N * H, TILE_N * H)])
    pltpu.sync_copy(acc_max.at[pl.ds(0, TILE_N * H)],
                    max_hbm.at[pl.ds(w * TILE_N * H, TILE_N * H)])


def _agg(m, sel_pk, counts):
    mesh = plsc.VectorSubcoreMesh(core_axis_name="c", subcore_axis_name="s")
    f = pl.kernel(
        _agg_body,
        out_type=[
            jax.ShapeDtypeStruct((NPAD * H,), jnp.float32),
            jax.ShapeDtypeStruct((NPAD * H,), jnp.float32),
        ],
        mesh=mesh,
        scratch_types=[
            pltpu.VMEM((CAP,), jnp.int32),
            pltpu.VMEM((2, CS), jnp.int32),
            pltpu.VMEM((2, CS, H), jnp.float32),
            pltpu.VMEM((16,), jnp.int32),
            pltpu.VMEM(((TILE_N + 1) * H,), jnp.float32),
            pltpu.VMEM(((TILE_N + 1) * H,), jnp.float32),
            pltpu.SemaphoreType.DMA((2,)),
        ],
        compiler_params=pltpu.CompilerParams(needs_layout_passes=False),
    )
    return f(m, sel_pk, counts)


def _mlp1_body(x_ref, w1_ref, b1_ref, w2_ref, b2_ref, w3_ref, b3_ref,
               wm_ref, bm_ref, feats_ref, m0_ref):
    h = jax.nn.relu(x_ref[...] @ w1_ref[...] + b1_ref[...])
    h = jax.nn.relu(h @ w2_ref[...] + b2_ref[...])
    f = h @ w3_ref[...] + b3_ref[...]
    feats_ref[...] = f
    wm = wm_ref[...]
    m0_ref[...] = jax.nn.relu(f @ (wm[:H, :] + wm[H:, :]) + bm_ref[...])


def _mlp1(x, w1, b1, w2, b2, w3, b3, wm, bm):
    grid = (N // _ROWS,)
    row_spec = pl.BlockSpec((_ROWS, H), lambda i: (i, 0))
    full = lambda a: pl.BlockSpec(a.shape, lambda i: (0,) * a.ndim)
    return pl.pallas_call(
        _mlp1_body,
        grid=grid,
        in_specs=[pl.BlockSpec((_ROWS, X), lambda i: (i, 0)),
                  full(w1), full(b1), full(w2), full(b2), full(w3), full(b3),
                  full(wm), full(bm)],
        out_specs=[row_spec, row_spec],
        out_shape=[jax.ShapeDtypeStruct((N, H), jnp.float32),
                   jax.ShapeDtypeStruct((N, H), jnp.float32)],
    )(x, w1, b1, w2, b2, w3, b3, wm, bm)


def _upd_common(res_ref, s_ref, mx_ref, cnt_ref, wu_ref, bu_ref):
    res = res_ref[...]
    s = s_ref[...]
    mx = mx_ref[...]
    cnt = cnt_ref[...]
    mean = s / jnp.maximum(cnt, 1.0)
    amp = jnp.log(cnt + 1.0) / DELTA
    wu = wu_ref[...]
    acc = (res @ wu[0:H, :] + mean @ wu[H:2 * H, :] + mx @ wu[2 * H:3 * H, :]
           + s @ wu[3 * H:4 * H, :] + (mean * amp) @ wu[4 * H:5 * H, :]
           + (mx * amp) @ wu[5 * H:6 * H, :] + (s * amp) @ wu[6 * H:7 * H, :]
           + bu_ref[...])
    return acc + res


def _upd0_body(res_ref, s_ref, mx_ref, cnt_ref, wu_ref, bu_ref,
               feats_ref, wm_ref, bm_ref, out_ref, m1_ref):
    out = _upd_common(res_ref, s_ref, mx_ref, cnt_ref, wu_ref, bu_ref)
    out_ref[...] = out
    wm = wm_ref[...]
    m1_ref[...] = jax.nn.relu(
        out @ wm[:H, :] + feats_ref[...] @ wm[H:, :] + bm_ref[...])


def _upd1_body(res_ref, s_ref, mx_ref, cnt_ref, wu_ref, bu_ref, out_ref):
    out_ref[...] = _upd_common(res_ref, s_ref, mx_ref, cnt_ref, wu_ref, bu_ref)


def _update0(res, s, mx, cnt, wu, bu, feats, wm, bm):
    grid = (N // _ROWS,)
    row_spec = pl.BlockSpec((_ROWS, H), lambda i: (i, 0))
    full = lambda a: pl.BlockSpec(a.shape, lambda i: (0,) * a.ndim)
    return pl.pallas_call(
        _upd0_body,
        grid=grid,
        in_specs=[row_spec, row_spec, row_spec,
                  pl.BlockSpec((_ROWS, 1), lambda i: (i, 0)),
                  full(wu), full(bu), row_spec, full(wm), full(bm)],
        out_specs=[row_spec, row_spec],
        out_shape=[jax.ShapeDtypeStruct((N, H), jnp.float32),
                   jax.ShapeDtypeStruct((N, H), jnp.float32)],
    )(res, s, mx, cnt, wu, bu, feats, wm, bm)


def _update1(res, s, mx, cnt, wu, bu):
    grid = (N // _ROWS,)
    row_spec = pl.BlockSpec((_ROWS, H), lambda i: (i, 0))
    full = lambda a: pl.BlockSpec(a.shape, lambda i: (0,) * a.ndim)
    return pl.pallas_call(
        _upd1_body,
        grid=grid,
        in_specs=[row_spec, row_spec, row_spec,
                  pl.BlockSpec((_ROWS, 1), lambda i: (i, 0)),
                  full(wu), full(bu)],
        out_specs=row_spec,
        out_shape=jax.ShapeDtypeStruct((N, H), jnp.float32),
    )(res, s, mx, cnt, wu, bu)


def _pool_body(res_ref, w_ref, b_ref, out_ref):
    pooled = jnp.sum(res_ref[...], axis=0, keepdims=True)
    out_ref[...] = pooled @ w_ref[...] + b_ref[...]


def _pool(res, w, b):
    return pl.pallas_call(
        _pool_body,
        out_shape=jax.ShapeDtypeStruct((1, C), jnp.float32),
    )(res, w, b)


def kernel(x, edge_index,
           mlp1_W1, mlp1_b1, mlp1_W2, mlp1_b2, mlp1_W3, mlp1_b3,
           conv0_Wm, conv0_bm, conv0_Wu, conv0_bu,
           conv1_Wm, conv1_bm, conv1_Wu, conv1_bu,
           lin_W, lin_b):
    b1 = mlp1_b1.reshape(1, -1)
    b2 = mlp1_b2.reshape(1, -1)
    b3 = mlp1_b3.reshape(1, -1)
    bm0 = conv0_bm.reshape(1, -1)
    bu0 = conv0_bu.reshape(1, -1)
    bm1 = conv1_bm.reshape(1, -1)
    bu1 = conv1_bu.reshape(1, -1)
    lb = lin_b.reshape(1, -1)

    feats, m0 = _mlp1(x, mlp1_W1, b1, mlp1_W2, b2, mlp1_W3, b3,
                      conv0_Wm, bm0)

    sel_pk, counts, cnt_f = _route(edge_index[0], edge_index[1])
    cnt = cnt_f.reshape(-1)[:N].reshape(N, 1)

    s0, mx0 = _agg(m0, sel_pk, counts)
    s0 = s0.reshape(NPAD, H)[:N]
    mx0 = mx0.reshape(NPAD, H)[:N]
    res1, m1 = _update0(feats, s0, mx0, cnt, conv0_Wu, bu0,
                        feats, conv1_Wm, bm1)

    s1, mx1 = _agg(m1, sel_pk, counts)
    s1 = s1.reshape(NPAD, H)[:N]
    mx1 = mx1.reshape(NPAD, H)[:N]
    res2 = _update1(res1, s1, mx1, cnt, conv1_Wu, bu1)

    return _pool(res2, lin_W, lb)


# speedup vs baseline: 1.5482x; 1.0610x over previous
import functools

import jax
import jax.numpy as jnp
from jax import lax
from jax.experimental import pallas as pl
from jax.experimental.pallas import tpu as pltpu
from jax.experimental.pallas import tpu_sc as plsc

N = 10000
E = 320000
X = 128
H = 128
C = 10
DELTA = 2.5

NC = 2
NS = 16
NW = NC * NS
TILE_N = 320
NPAD = NW * TILE_N
CAP = 16384
CE = 3200
CS = 112
HL = H // 16

_ROWS = 1000


def _route_body(esrc_hbm, edst_hbm, selpk_hbm, counts_hbm, cntf_hbm,
                src_v, dst_v, selpk_v, sorted_v, cvec_v, hist_fv,
                hist_s, start_s, esem):
    w = lax.axis_index("s") * NC + lax.axis_index("c")
    lo = w * TILE_N
    zero16 = jnp.zeros((16,), jnp.int32)
    sent16 = jnp.full((16,), TILE_N, jnp.int32)
    NG = CE // 16

    def fire_e(c, b):
        pltpu.async_copy(esrc_hbm.at[pl.ds(c * CE, CE)],
                         src_v.at[pl.ds(b * CE, CE)], esem.at[b])
        pltpu.async_copy(edst_hbm.at[pl.ds(c * CE, CE)],
                         dst_v.at[pl.ds(b * CE, CE)], esem.at[b])

    def wait_e(c, b):
        pltpu.make_async_copy(esrc_hbm.at[pl.ds(c * CE, CE)],
                              src_v.at[pl.ds(b * CE, CE)], esem.at[b]).wait()
        pltpu.make_async_copy(edst_hbm.at[pl.ds(c * CE, CE)],
                              dst_v.at[pl.ds(b * CE, CE)], esem.at[b]).wait()

    fire_e(0, 0)

    def chunk(cb, off_v):
        b = cb % 2

        @pl.when(cb + 1 < E // CE)
        def _():
            fire_e(cb + 1, 1 - b)

        wait_e(cb, b)

        def step(g, off_v):
            base = b * CE + g * 64
            dls, ss, ms, ps, ns, vs = [], [], [], [], [], []
            for u in range(4):
                dl = dst_v[pl.ds(base + u * 16, 16)] - lo
                s16 = src_v[pl.ds(base + u * 16, 16)]
                m = (dl >= 0) & (dl < TILE_N)
                dls.append(dl)
                ss.append(s16)
                ms.append(m)
            for u in range(4):
                ps.append(plsc.cumsum(jnp.where(ms[u], 1, 0)))
                ns.append(plsc.all_reduce_population_count(ms[u]))
                vs.append(jnp.left_shift(ss[u], 9) | (dls[u] & 511))
            run = off_v
            for u in range(4):
                plsc.store_scatter(selpk_v, [run + ps[u] - 1], vs[u],
                                   mask=ms[u])
                run = run + ns[u]
            return run

        return lax.fori_loop(0, NG // 4, step, off_v)

    off_v = lax.fori_loop(0, E // CE, chunk, zero16)
    count = off_v[0]

    def pad(i, _):
        selpk_v[pl.ds(count + i * 16, 16)] = sent16
        return 0

    lax.fori_loop(0, CS // 16, pad, 0)

    cvec_v[...] = zero16 + count
    pltpu.sync_copy(cvec_v, counts_hbm.at[w])

    def hzero(i, _):
        hist_s[i] = 0
        return 0

    lax.fori_loop(0, TILE_N + 1, hzero, 0)

    ngrp = ((count + CS - 1) // CS) * (CS // 16)

    def hstep(gi, _):
        v = selpk_v[pl.ds(gi * 16, 16)]
        for l in range(16):
            d = v[l] & 511
            hist_s[d] = hist_s[d] + 1
        return 0

    lax.fori_loop(0, ngrp, hstep, 0)

    lanes = lax.iota(jnp.int32, 16)
    lane0 = lanes == 0

    def hout(i, _):
        hv = (zero16 + hist_s[i]).astype(jnp.float32)
        plsc.store_scatter(hist_fv, [zero16 + i], hv, mask=lane0)
        return 0

    lax.fori_loop(0, TILE_N, hout, 0)
    pltpu.sync_copy(hist_fv, cntf_hbm.at[w])

    def pfx(d, run):
        start_s[d] = run
        return run + hist_s[d]

    lax.fori_loop(0, TILE_N + 1, pfx, jnp.int32(0))

    def rstep(gi, _):
        v = selpk_v[pl.ds(gi * 16, 16)]
        for l in range(16):
            pk = v[l]
            d = pk & 511
            slot = start_s[d]
            start_s[d] = slot + 1
            plsc.store_scatter(sorted_v, [zero16 + slot], zero16 + pk,
                               mask=lane0)
        return 0

    lax.fori_loop(0, (count + 15) // 16, rstep, 0)

    def pad2(i, _):
        sorted_v[pl.ds(count + i * 16, 16)] = sent16
        return 0

    lax.fori_loop(0, CS // 16, pad2, 0)
    pltpu.sync_copy(sorted_v, selpk_hbm.at[w])


def _route(esrc, edst):
    mesh = plsc.VectorSubcoreMesh(core_axis_name="c", subcore_axis_name="s")
    f = pl.kernel(
        _route_body,
        out_type=[
            jax.ShapeDtypeStruct((NW, CAP), jnp.int32),
            jax.ShapeDtypeStruct((NW, 16), jnp.int32),
            jax.ShapeDtypeStruct((NW, TILE_N), jnp.float32),
        ],
        mesh=mesh,
        scratch_types=[
            pltpu.VMEM((2 * CE,), jnp.int32),
            pltpu.VMEM((2 * CE,), jnp.int32),
            pltpu.VMEM((CAP,), jnp.int32),
            pltpu.VMEM((CAP,), jnp.int32),
            pltpu.VMEM((16,), jnp.int32),
            pltpu.VMEM((TILE_N,), jnp.float32),
            pltpu.SMEM((TILE_N + 8,), jnp.int32),
            pltpu.SMEM((TILE_N + 8,), jnp.int32),
            pltpu.SemaphoreType.DMA((2,)),
        ],
        compiler_params=pltpu.CompilerParams(needs_layout_passes=False),
    )
    return f(esrc, edst)


def _agg_body(m_hbm, selpk_hbm, counts_hbm, sum_hbm, max_hbm,
              pkall_v, src_v, rows_v, cvec_v, acc_sum, acc_max, gsem):
    w = lax.axis_index("s") * NC + lax.axis_index("c")
    z16 = jnp.zeros((16,), jnp.float32)

    def zr(i, _):
        acc_sum[pl.ds(i * 16, 16)] = z16
        acc_max[pl.ds(i * 16, 16)] = z16
        return 0

    lax.fori_loop(0, (TILE_N + 1) * H // 16, zr, 0)

    pltpu.sync_copy(selpk_hbm.at[w], pkall_v)
    pltpu.sync_copy(counts_hbm.at[w], cvec_v)
    count = cvec_v[...][0]
    nch = (count + CS - 1) // CS

    def fire(c, b):
        def up(g, _):
            v = pkall_v[pl.ds(c * CS + g * 16, 16)]
            src_v[b, pl.ds(g * 16, 16)] = jnp.right_shift(v, 9)
            return 0

        lax.fori_loop(0, CS // 16, up, 0)
        pltpu.async_copy(m_hbm.at[src_v.at[b]], rows_v.at[b], gsem.at[b])

    def wait_g(b):
        pltpu.make_async_copy(m_hbm.at[src_v.at[b]], rows_v.at[b],
                              gsem.at[b]).wait()

    @pl.when(nch > 0)
    def _():
        fire(0, 0)

    def flush(cur_d, regs):
        off = cur_d * H
        for j in range(HL):
            acc_sum[pl.ds(off + j * 16, 16)] = regs[j]
            acc_max[pl.ds(off + j * 16, 16)] = regs[HL + j]

    z16f = jnp.zeros((16,), jnp.float32)
    zregs = (z16f,) * (2 * HL)

    def chunk(cb, carry):
        b = cb % 2

        @pl.when(cb + 1 < nch)
        def _():
            fire(cb + 1, 1 - b)

        wait_g(b)

        def grp(gl, carry):
            dlv = pkall_v[pl.ds(cb * CS + gl * 16, 16)]
            for l in range(16):
                d = dlv[l] & 511
                cur_d, regs = carry[0], carry[1:]

                def on_change(cd, rs):
                    flush(cd, rs)
                    return (d,) + zregs

                def keep(cd, rs):
                    return (cd,) + rs

                carry = lax.cond(d != cur_d, on_change, keep, cur_d, regs)
                row = [rows_v[b, gl * 16 + l, pl.ds(j * 16, 16)]
                       for j in range(HL)]
                carry = (carry[0],) + tuple(
                    carry[1 + j] + row[j] for j in range(HL)) + tuple(
                    jnp.maximum(carry[1 + HL + j], row[j]) for j in range(HL))
            return carry

        return lax.fori_loop(0, CS // 16, grp, carry)

    fcarry = lax.fori_loop(0, nch, chunk, (jnp.int32(TILE_N),) + zregs)
    flush(fcarry[0], fcarry[1:])

    pltpu.sync_copy(acc_sum.at[pl.ds(0, TILE_N * H)],
                    sum_hbm.at[pl.ds(w * TILE_N * H, TILE_N * H)])
    pltpu.sync_copy(acc_max.at[pl.ds(0, TILE_N * H)],
                    max_hbm.at[pl.ds(w * TILE_N * H, TILE_N * H)])


def _agg(m, sel_pk, counts):
    mesh = plsc.VectorSubcoreMesh(core_axis_name="c", subcore_axis_name="s")
    f = pl.kernel(
        _agg_body,
        out_type=[
            jax.ShapeDtypeStruct((NPAD * H,), jnp.float32),
            jax.ShapeDtypeStruct((NPAD * H,), jnp.float32),
        ],
        mesh=mesh,
        scratch_types=[
            pltpu.VMEM((CAP,), jnp.int32),
            pltpu.VMEM((2, CS), jnp.int32),
            pltpu.VMEM((2, CS, H), jnp.float32),
            pltpu.VMEM((16,), jnp.int32),
            pltpu.VMEM(((TILE_N + 1) * H,), jnp.float32),
            pltpu.VMEM(((TILE_N + 1) * H,), jnp.float32),
            pltpu.SemaphoreType.DMA((2,)),
        ],
        compiler_params=pltpu.CompilerParams(needs_layout_passes=False),
    )
    return f(m, sel_pk, counts)


def _mlp1_body(x_ref, w1_ref, b1_ref, w2_ref, b2_ref, w3_ref, b3_ref,
               wm_ref, bm_ref, feats_ref, m0_ref):
    h = jax.nn.relu(x_ref[...] @ w1_ref[...] + b1_ref[...])
    h = jax.nn.relu(h @ w2_ref[...] + b2_ref[...])
    f = h @ w3_ref[...] + b3_ref[...]
    feats_ref[...] = f
    wm = wm_ref[...]
    m0_ref[...] = jax.nn.relu(f @ (wm[:H, :] + wm[H:, :]) + bm_ref[...])


def _mlp1(x, w1, b1, w2, b2, w3, b3, wm, bm):
    grid = (N // _ROWS,)
    row_spec = pl.BlockSpec((_ROWS, H), lambda i: (i, 0))
    full = lambda a: pl.BlockSpec(a.shape, lambda i: (0,) * a.ndim)
    return pl.pallas_call(
        _mlp1_body,
        grid=grid,
        in_specs=[pl.BlockSpec((_ROWS, X), lambda i: (i, 0)),
                  full(w1), full(b1), full(w2), full(b2), full(w3), full(b3),
                  full(wm), full(bm)],
        out_specs=[row_spec, row_spec],
        out_shape=[jax.ShapeDtypeStruct((N, H), jnp.float32),
                   jax.ShapeDtypeStruct((N, H), jnp.float32)],
    )(x, w1, b1, w2, b2, w3, b3, wm, bm)


def _upd_common(res_ref, s_ref, mx_ref, cnt_ref, wu_ref, bu_ref):
    res = res_ref[...]
    s = s_ref[...]
    mx = mx_ref[...]
    cnt = cnt_ref[...]
    mean = s / jnp.maximum(cnt, 1.0)
    amp = jnp.log(cnt + 1.0) / DELTA
    wu = wu_ref[...]
    acc = (res @ wu[0:H, :] + mean @ wu[H:2 * H, :] + mx @ wu[2 * H:3 * H, :]
           + s @ wu[3 * H:4 * H, :] + (mean * amp) @ wu[4 * H:5 * H, :]
           + (mx * amp) @ wu[5 * H:6 * H, :] + (s * amp) @ wu[6 * H:7 * H, :]
           + bu_ref[...])
    return acc + res


def _upd0_body(res_ref, s_ref, mx_ref, cnt_ref, wu_ref, bu_ref,
               feats_ref, wm_ref, bm_ref, out_ref, m1_ref):
    out = _upd_common(res_ref, s_ref, mx_ref, cnt_ref, wu_ref, bu_ref)
    out_ref[...] = out
    wm = wm_ref[...]
    m1_ref[...] = jax.nn.relu(
        out @ wm[:H, :] + feats_ref[...] @ wm[H:, :] + bm_ref[...])


def _upd1_body(res_ref, s_ref, mx_ref, cnt_ref, wu_ref, bu_ref, out_ref):
    out_ref[...] = _upd_common(res_ref, s_ref, mx_ref, cnt_ref, wu_ref, bu_ref)


def _update0(res, s, mx, cnt, wu, bu, feats, wm, bm):
    grid = (N // _ROWS,)
    row_spec = pl.BlockSpec((_ROWS, H), lambda i: (i, 0))
    full = lambda a: pl.BlockSpec(a.shape, lambda i: (0,) * a.ndim)
    return pl.pallas_call(
        _upd0_body,
        grid=grid,
        in_specs=[row_spec, row_spec, row_spec,
                  pl.BlockSpec((_ROWS, 1), lambda i: (i, 0)),
                  full(wu), full(bu), row_spec, full(wm), full(bm)],
        out_specs=[row_spec, row_spec],
        out_shape=[jax.ShapeDtypeStruct((N, H), jnp.float32),
                   jax.ShapeDtypeStruct((N, H), jnp.float32)],
    )(res, s, mx, cnt, wu, bu, feats, wm, bm)


def _update1(res, s, mx, cnt, wu, bu):
    grid = (N // _ROWS,)
    row_spec = pl.BlockSpec((_ROWS, H), lambda i: (i, 0))
    full = lambda a: pl.BlockSpec(a.shape, lambda i: (0,) * a.ndim)
    return pl.pallas_call(
        _upd1_body,
        grid=grid,
        in_specs=[row_spec, row_spec, row_spec,
                  pl.BlockSpec((_ROWS, 1), lambda i: (i, 0)),
                  full(wu), full(bu)],
        out_specs=row_spec,
        out_shape=jax.ShapeDtypeStruct((N, H), jnp.float32),
    )(res, s, mx, cnt, wu, bu)


def _pool_body(res_ref, w_ref, b_ref, out_ref):
    pooled = jnp.sum(res_ref[...], axis=0, keepdims=True)
    out_ref[...] = pooled @ w_ref[...] + b_ref[...]


def _pool(res, w, b):
    return pl.pallas_call(
        _pool_body,
        out_shape=jax.ShapeDtypeStruct((1, C), jnp.float32),
    )(res, w, b)


def kernel(x, edge_index,
           mlp1_W1, mlp1_b1, mlp1_W2, mlp1_b2, mlp1_W3, mlp1_b3,
           conv0_Wm, conv0_bm, conv0_Wu, conv0_bu,
           conv1_Wm, conv1_bm, conv1_Wu, conv1_bu,
           lin_W, lin_b):
    b1 = mlp1_b1.reshape(1, -1)
    b2 = mlp1_b2.reshape(1, -1)
    b3 = mlp1_b3.reshape(1, -1)
    bm0 = conv0_bm.reshape(1, -1)
    bu0 = conv0_bu.reshape(1, -1)
    bm1 = conv1_bm.reshape(1, -1)
    bu1 = conv1_bu.reshape(1, -1)
    lb = lin_b.reshape(1, -1)

    feats, m0 = _mlp1(x, mlp1_W1, b1, mlp1_W2, b2, mlp1_W3, b3,
                      conv0_Wm, bm0)

    sel_pk, counts, cnt_f = _route(edge_index[0], edge_index[1])
    cnt = cnt_f.reshape(-1)[:N].reshape(N, 1)

    s0, mx0 = _agg(m0, sel_pk, counts)
    s0 = s0.reshape(NPAD, H)[:N]
    mx0 = mx0.reshape(NPAD, H)[:N]
    res1, m1 = _update0(feats, s0, mx0, cnt, conv0_Wu, bu0,
                        feats, conv1_Wm, bm1)

    s1, mx1 = _agg(m1, sel_pk, counts)
    s1 = s1.reshape(NPAD, H)[:N]
    mx1 = mx1.reshape(NPAD, H)[:N]
    res2 = _update1(res1, s1, mx1, cnt, conv1_Wu, bu1)

    return _pool(res2, lin_W, lb)


# speedup vs baseline: 1.6661x; 1.0762x over previous
import functools

import jax
import jax.numpy as jnp
from jax import lax
from jax.experimental import pallas as pl
from jax.experimental.pallas import tpu as pltpu
from jax.experimental.pallas import tpu_sc as plsc

N = 10000
E = 320000
X = 128
H = 128
C = 10
DELTA = 2.5

NC = 2
NS = 16
NW = NC * NS
TILE_N = 320
NPAD = NW * TILE_N
CAP = 16384
CE = 3200
CS = 112
CA = 80
HL = H // 16

_ROWS = 1000


def _route_body(esrc_hbm, edst_hbm, selpk_hbm, counts_hbm, cntf_hbm,
                src_v, dst_v, selpk_v, sorted_v, cvec_v, hist_fv,
                hist_s, start_s, esem):
    w = lax.axis_index("s") * NC + lax.axis_index("c")
    lo = w * TILE_N
    zero16 = jnp.zeros((16,), jnp.int32)
    sent16 = jnp.full((16,), TILE_N, jnp.int32)
    NG = CE // 16

    def fire_e(c, b):
        pltpu.async_copy(esrc_hbm.at[pl.ds(c * CE, CE)],
                         src_v.at[pl.ds(b * CE, CE)], esem.at[b])
        pltpu.async_copy(edst_hbm.at[pl.ds(c * CE, CE)],
                         dst_v.at[pl.ds(b * CE, CE)], esem.at[b])

    def wait_e(c, b):
        pltpu.make_async_copy(esrc_hbm.at[pl.ds(c * CE, CE)],
                              src_v.at[pl.ds(b * CE, CE)], esem.at[b]).wait()
        pltpu.make_async_copy(edst_hbm.at[pl.ds(c * CE, CE)],
                              dst_v.at[pl.ds(b * CE, CE)], esem.at[b]).wait()

    fire_e(0, 0)

    def chunk(cb, off_v):
        b = cb % 2

        @pl.when(cb + 1 < E // CE)
        def _():
            fire_e(cb + 1, 1 - b)

        wait_e(cb, b)

        def step(g, off_v):
            base = b * CE + g * 64
            dls, ss, ms, ps, ns, vs = [], [], [], [], [], []
            for u in range(4):
                dl = dst_v[pl.ds(base + u * 16, 16)] - lo
                s16 = src_v[pl.ds(base + u * 16, 16)]
                m = (dl >= 0) & (dl < TILE_N)
                dls.append(dl)
                ss.append(s16)
                ms.append(m)
            for u in range(4):
                ps.append(plsc.cumsum(jnp.where(ms[u], 1, 0)))
                ns.append(plsc.all_reduce_population_count(ms[u]))
                vs.append(jnp.left_shift(ss[u], 9) | (dls[u] & 511))
            run = off_v
            for u in range(4):
                plsc.store_scatter(selpk_v, [run + ps[u] - 1], vs[u],
                                   mask=ms[u])
                run = run + ns[u]
            return run

        return lax.fori_loop(0, NG // 4, step, off_v)

    off_v = lax.fori_loop(0, E // CE, chunk, zero16)
    count = off_v[0]

    def pad(i, _):
        selpk_v[pl.ds(count + i * 16, 16)] = sent16
        return 0

    lax.fori_loop(0, CS // 16, pad, 0)

    cvec_v[...] = zero16 + count
    pltpu.sync_copy(cvec_v, counts_hbm.at[w])

    def hzero(i, _):
        hist_s[i] = 0
        return 0

    lax.fori_loop(0, TILE_N + 1, hzero, 0)

    ngrp = ((count + CS - 1) // CS) * (CS // 16)

    def hstep(gi, _):
        v = selpk_v[pl.ds(gi * 16, 16)]
        for l in range(16):
            d = v[l] & 511
            hist_s[d] = hist_s[d] + 1
        return 0

    lax.fori_loop(0, ngrp, hstep, 0)

    lanes = lax.iota(jnp.int32, 16)
    lane0 = lanes == 0

    def hout(i, _):
        hv = (zero16 + hist_s[i]).astype(jnp.float32)
        plsc.store_scatter(hist_fv, [zero16 + i], hv, mask=lane0)
        return 0

    lax.fori_loop(0, TILE_N, hout, 0)
    pltpu.sync_copy(hist_fv, cntf_hbm.at[w])

    def pfx(d, run):
        start_s[d] = run
        return run + hist_s[d]

    lax.fori_loop(0, TILE_N + 1, pfx, jnp.int32(0))

    def rstep(gi, _):
        v = selpk_v[pl.ds(gi * 16, 16)]
        for l in range(16):
            pk = v[l]
            d = pk & 511
            slot = start_s[d]
            start_s[d] = slot + 1
            plsc.store_scatter(sorted_v, [zero16 + slot], zero16 + pk,
                               mask=lane0)
        return 0

    lax.fori_loop(0, (count + 15) // 16, rstep, 0)

    def pad2(i, _):
        sorted_v[pl.ds(count + i * 16, 16)] = sent16
        return 0

    lax.fori_loop(0, CS // 16, pad2, 0)
    pltpu.sync_copy(sorted_v, selpk_hbm.at[w])


def _route(esrc, edst):
    mesh = plsc.VectorSubcoreMesh(core_axis_name="c", subcore_axis_name="s")
    f = pl.kernel(
        _route_body,
        out_type=[
            jax.ShapeDtypeStruct((NW, CAP), jnp.int32),
            jax.ShapeDtypeStruct((NW, 16), jnp.int32),
            jax.ShapeDtypeStruct((NW, TILE_N), jnp.float32),
        ],
        mesh=mesh,
        scratch_types=[
            pltpu.VMEM((2 * CE,), jnp.int32),
            pltpu.VMEM((2 * CE,), jnp.int32),
            pltpu.VMEM((CAP,), jnp.int32),
            pltpu.VMEM((CAP,), jnp.int32),
            pltpu.VMEM((16,), jnp.int32),
            pltpu.VMEM((TILE_N,), jnp.float32),
            pltpu.SMEM((TILE_N + 8,), jnp.int32),
            pltpu.SMEM((TILE_N + 8,), jnp.int32),
            pltpu.SemaphoreType.DMA((2,)),
        ],
        compiler_params=pltpu.CompilerParams(needs_layout_passes=False),
    )
    return f(esrc, edst)


def _agg_body(m_hbm, selpk_hbm, counts_hbm, sum_hbm, max_hbm,
              pkall_v, src_v, rows_v, cvec_v, acc_sum, acc_max, gsem):
    w = lax.axis_index("s") * NC + lax.axis_index("c")
    z16 = jnp.zeros((16,), jnp.float32)

    def zr(i, _):
        acc_sum[pl.ds(i * 16, 16)] = z16
        acc_max[pl.ds(i * 16, 16)] = z16
        return 0

    lax.fori_loop(0, (TILE_N + 1) * H // 16, zr, 0)

    pltpu.sync_copy(selpk_hbm.at[w], pkall_v)
    pltpu.sync_copy(counts_hbm.at[w], cvec_v)
    count = cvec_v[...][0]
    nch = (count + CA - 1) // CA

    def fire(c, b):
        def up(g, _):
            v = pkall_v[pl.ds(c * CA + g * 16, 16)]
            src_v[b, pl.ds(g * 16, 16)] = jnp.right_shift(v, 9)
            return 0

        lax.fori_loop(0, CA // 16, up, 0)
        pltpu.async_copy(m_hbm.at[src_v.at[b]], rows_v.at[b], gsem.at[b])

    def wait_g(b):
        pltpu.make_async_copy(m_hbm.at[src_v.at[b]], rows_v.at[b],
                              gsem.at[b]).wait()

    @pl.when(nch > 0)
    def _():
        fire(0, 0)

    @pl.when(nch > 1)
    def _():
        fire(1, 1)

    def flush(cur_d, regs):
        off = cur_d * H
        for j in range(HL):
            acc_sum[pl.ds(off + j * 16, 16)] = regs[j]
            acc_max[pl.ds(off + j * 16, 16)] = regs[HL + j]

    z16f = jnp.zeros((16,), jnp.float32)
    zregs = (z16f,) * (2 * HL)

    def chunk(cb, carry):
        b = cb % 3

        @pl.when(cb + 2 < nch)
        def _():
            fire(cb + 2, (cb + 2) % 3)

        wait_g(b)

        def grp(gl, carry):
            dlv = pkall_v[pl.ds(cb * CA + gl * 16, 16)]
            for l in range(16):
                d = dlv[l] & 511
                cur_d, regs = carry[0], carry[1:]

                def on_change(cd, rs):
                    flush(cd, rs)
                    return (d,) + zregs

                def keep(cd, rs):
                    return (cd,) + rs

                carry = lax.cond(d != cur_d, on_change, keep, cur_d, regs)
                row = [rows_v[b, gl * 16 + l, pl.ds(j * 16, 16)]
                       for j in range(HL)]
                carry = (carry[0],) + tuple(
                    carry[1 + j] + row[j] for j in range(HL)) + tuple(
                    jnp.maximum(carry[1 + HL + j], row[j]) for j in range(HL))
            return carry

        return lax.fori_loop(0, CA // 16, grp, carry)

    fcarry = lax.fori_loop(0, nch, chunk, (jnp.int32(TILE_N),) + zregs)
    flush(fcarry[0], fcarry[1:])

    pltpu.sync_copy(acc_sum.at[pl.ds(0, TILE_N * H)],
                    sum_hbm.at[pl.ds(w * TILE_N * H, TILE_N * H)])
    pltpu.sync_copy(acc_max.at[pl.ds(0, TILE_N * H)],
                    max_hbm.at[pl.ds(w * TILE_N * H, TILE_N * H)])


def _agg(m, sel_pk, counts):
    mesh = plsc.VectorSubcoreMesh(core_axis_name="c", subcore_axis_name="s")
    f = pl.kernel(
        _agg_body,
        out_type=[
            jax.ShapeDtypeStruct((NPAD * H,), jnp.float32),
            jax.ShapeDtypeStruct((NPAD * H,), jnp.float32),
        ],
        mesh=mesh,
        scratch_types=[
            pltpu.VMEM((CAP,), jnp.int32),
            pltpu.VMEM((3, CA), jnp.int32),
            pltpu.VMEM((3, CA, H), jnp.float32),
            pltpu.VMEM((16,), jnp.int32),
            pltpu.VMEM(((TILE_N + 1) * H,), jnp.float32),
            pltpu.VMEM(((TILE_N + 1) * H,), jnp.float32),
            pltpu.SemaphoreType.DMA((3,)),
        ],
        compiler_params=pltpu.CompilerParams(needs_layout_passes=False),
    )
    return f(m, sel_pk, counts)


def _mlp1_body(x_ref, w1_ref, b1_ref, w2_ref, b2_ref, w3_ref, b3_ref,
               wm_ref, bm_ref, feats_ref, m0_ref):
    h = jax.nn.relu(x_ref[...] @ w1_ref[...] + b1_ref[...])
    h = jax.nn.relu(h @ w2_ref[...] + b2_ref[...])
    f = h @ w3_ref[...] + b3_ref[...]
    feats_ref[...] = f
    wm = wm_ref[...]
    m0_ref[...] = jax.nn.relu(f @ (wm[:H, :] + wm[H:, :]) + bm_ref[...])


def _mlp1(x, w1, b1, w2, b2, w3, b3, wm, bm):
    grid = (N // _ROWS,)
    row_spec = pl.BlockSpec((_ROWS, H), lambda i: (i, 0))
    full = lambda a: pl.BlockSpec(a.shape, lambda i: (0,) * a.ndim)
    return pl.pallas_call(
        _mlp1_body,
        grid=grid,
        in_specs=[pl.BlockSpec((_ROWS, X), lambda i: (i, 0)),
                  full(w1), full(b1), full(w2), full(b2), full(w3), full(b3),
                  full(wm), full(bm)],
        out_specs=[row_spec, row_spec],
        out_shape=[jax.ShapeDtypeStruct((N, H), jnp.float32),
                   jax.ShapeDtypeStruct((N, H), jnp.float32)],
    )(x, w1, b1, w2, b2, w3, b3, wm, bm)


def _upd_common(res_ref, s_ref, mx_ref, cnt_ref, wu_ref, bu_ref):
    res = res_ref[...]
    s = s_ref[...]
    mx = mx_ref[...]
    cnt = cnt_ref[...]
    mean = s / jnp.maximum(cnt, 1.0)
    amp = jnp.log(cnt + 1.0) / DELTA
    wu = wu_ref[...]
    acc = (res @ wu[0:H, :] + mean @ wu[H:2 * H, :] + mx @ wu[2 * H:3 * H, :]
           + s @ wu[3 * H:4 * H, :] + (mean * amp) @ wu[4 * H:5 * H, :]
           + (mx * amp) @ wu[5 * H:6 * H, :] + (s * amp) @ wu[6 * H:7 * H, :]
           + bu_ref[...])
    return acc + res


def _upd0_body(res_ref, s_ref, mx_ref, cnt_ref, wu_ref, bu_ref,
               feats_ref, wm_ref, bm_ref, out_ref, m1_ref):
    out = _upd_common(res_ref, s_ref, mx_ref, cnt_ref, wu_ref, bu_ref)
    out_ref[...] = out
    wm = wm_ref[...]
    m1_ref[...] = jax.nn.relu(
        out @ wm[:H, :] + feats_ref[...] @ wm[H:, :] + bm_ref[...])


def _upd1_body(res_ref, s_ref, mx_ref, cnt_ref, wu_ref, bu_ref, out_ref):
    out_ref[...] = _upd_common(res_ref, s_ref, mx_ref, cnt_ref, wu_ref, bu_ref)


def _update0(res, s, mx, cnt, wu, bu, feats, wm, bm):
    grid = (N // _ROWS,)
    row_spec = pl.BlockSpec((_ROWS, H), lambda i: (i, 0))
    full = lambda a: pl.BlockSpec(a.shape, lambda i: (0,) * a.ndim)
    return pl.pallas_call(
        _upd0_body,
        grid=grid,
        in_specs=[row_spec, row_spec, row_spec,
                  pl.BlockSpec((_ROWS, 1), lambda i: (i, 0)),
                  full(wu), full(bu), row_spec, full(wm), full(bm)],
        out_specs=[row_spec, row_spec],
        out_shape=[jax.ShapeDtypeStruct((N, H), jnp.float32),
                   jax.ShapeDtypeStruct((N, H), jnp.float32)],
    )(res, s, mx, cnt, wu, bu, feats, wm, bm)


def _update1(res, s, mx, cnt, wu, bu):
    grid = (N // _ROWS,)
    row_spec = pl.BlockSpec((_ROWS, H), lambda i: (i, 0))
    full = lambda a: pl.BlockSpec(a.shape, lambda i: (0,) * a.ndim)
    return pl.pallas_call(
        _upd1_body,
        grid=grid,
        in_specs=[row_spec, row_spec, row_spec,
                  pl.BlockSpec((_ROWS, 1), lambda i: (i, 0)),
                  full(wu), full(bu)],
        out_specs=row_spec,
        out_shape=jax.ShapeDtypeStruct((N, H), jnp.float32),
    )(res, s, mx, cnt, wu, bu)


def _pool_body(res_ref, w_ref, b_ref, out_ref):
    pooled = jnp.sum(res_ref[...], axis=0, keepdims=True)
    out_ref[...] = pooled @ w_ref[...] + b_ref[...]


def _pool(res, w, b):
    return pl.pallas_call(
        _pool_body,
        out_shape=jax.ShapeDtypeStruct((1, C), jnp.float32),
    )(res, w, b)


def kernel(x, edge_index,
           mlp1_W1, mlp1_b1, mlp1_W2, mlp1_b2, mlp1_W3, mlp1_b3,
           conv0_Wm, conv0_bm, conv0_Wu, conv0_bu,
           conv1_Wm, conv1_bm, conv1_Wu, conv1_bu,
           lin_W, lin_b):
    b1 = mlp1_b1.reshape(1, -1)
    b2 = mlp1_b2.reshape(1, -1)
    b3 = mlp1_b3.reshape(1, -1)
    bm0 = conv0_bm.reshape(1, -1)
    bu0 = conv0_bu.reshape(1, -1)
    bm1 = conv1_bm.reshape(1, -1)
    bu1 = conv1_bu.reshape(1, -1)
    lb = lin_b.reshape(1, -1)

    feats, m0 = _mlp1(x, mlp1_W1, b1, mlp1_W2, b2, mlp1_W3, b3,
                      conv0_Wm, bm0)

    sel_pk, counts, cnt_f = _route(edge_index[0], edge_index[1])
    cnt = cnt_f.reshape(-1)[:N].reshape(N, 1)

    s0, mx0 = _agg(m0, sel_pk, counts)
    s0 = s0.reshape(NPAD, H)[:N]
    mx0 = mx0.reshape(NPAD, H)[:N]
    res1, m1 = _update0(feats, s0, mx0, cnt, conv0_Wu, bu0,
                        feats, conv1_Wm, bm1)

    s1, mx1 = _agg(m1, sel_pk, counts)
    s1 = s1.reshape(NPAD, H)[:N]
    mx1 = mx1.reshape(NPAD, H)[:N]
    res2 = _update1(res1, s1, mx1, cnt, conv1_Wu, bu1)

    return _pool(res2, lin_W, lb)
